# Initial kernel scaffold; baseline (speedup 1.0000x reference)
import jax
import jax.numpy as jnp
from jax.experimental import pallas as pl


def kernel(src, in_proj_w, in_proj_b, out_proj_w, out_proj_b, ln1_g, ln1_b, ln2_g, ln2_b, router_w, router_b, w1, b1, w2, b2):
    raise NotImplementedError("write your pallas kernel here")


# speedup vs baseline: 1.6250x; 1.6250x over previous
import functools

import jax
import jax.numpy as jnp
from jax.experimental import pallas as pl
from jax.experimental.pallas import tpu as pltpu


H = 12
E = 8


def _ln(y, g, b):
    m = jnp.mean(y, axis=1, keepdims=True)
    v = jnp.mean((y - m) ** 2, axis=1, keepdims=True)
    return (y - m) * jax.lax.rsqrt(v + 1e-5) * g + b


def _qkv_kernel(x_ref, w_ref, b_ref, o_ref):
    o_ref[...] = (
        jnp.dot(x_ref[...], w_ref[...], preferred_element_type=jnp.float32)
        + b_ref[...]
    )


def _attn_kernel(qrow_ref, kv_ref, o_ref, *, scale, dh):
    q_all = qrow_ref[...]
    kv_all = kv_ref[...]
    D = H * dh
    outs = []
    for h in range(H):
        qh = q_all[:, h * dh:(h + 1) * dh]
        kh = kv_all[:, D + h * dh:D + (h + 1) * dh]
        vh = kv_all[:, 2 * D + h * dh:2 * D + (h + 1) * dh]
        s = jax.lax.dot_general(
            qh, kh, (((1,), (1,)), ((), ())), preferred_element_type=jnp.float32
        ) * scale
        m = jnp.max(s, axis=1, keepdims=True)
        p = jnp.exp(s - m)
        l = jnp.sum(p, axis=1, keepdims=True)
        outs.append(
            jnp.dot(p / l, vh, preferred_element_type=jnp.float32))
    o_ref[...] = jnp.concatenate(outs, axis=1)


def _post_attn_kernel(ctx_ref, src_ref, wo_ref, bo_ref, g1_ref, b1_ref,
                      rw_ref, rb_ref, x_ref, gpe_ref):
    attn = (
        jnp.dot(ctx_ref[...], wo_ref[...], preferred_element_type=jnp.float32)
        + bo_ref[...]
    )
    x = _ln(src_ref[...] + attn, g1_ref[...], b1_ref[...])
    x_ref[...] = x
    logits = (
        jnp.dot(x, rw_ref[...], preferred_element_type=jnp.float32) + rb_ref[...]
    )
    ii = jax.lax.broadcasted_iota(jnp.int32, logits.shape, 1)
    m1 = jnp.max(logits, axis=1, keepdims=True)
    i1 = jnp.min(jnp.where(logits == m1, ii, E), axis=1, keepdims=True)
    masked = jnp.where(ii == i1, -jnp.inf, logits)
    m2 = jnp.max(masked, axis=1, keepdims=True)
    i2 = jnp.min(jnp.where(masked == m2, ii, E), axis=1, keepdims=True)
    gate1 = 1.0 / (1.0 + jnp.exp(m2 - m1))
    gpe_ref[...] = jnp.where(ii == i1, gate1, 0.0) + jnp.where(
        ii == i2, 1.0 - gate1, 0.0
    )


def _moe_kernel(x_ref, gpe_ref, w1_ref, b1_ref, w2_ref, b2_ref,
                g2_ref, bb2_ref, o_ref, acc_ref):
    e = pl.program_id(1)

    @pl.when(e == 0)
    def _():
        acc_ref[...] = jnp.zeros_like(acc_ref)

    x = x_ref[...]
    h = jnp.maximum(
        jnp.dot(x, w1_ref[0], preferred_element_type=jnp.float32) + b1_ref[0],
        0.0,
    )
    y = jnp.dot(h, w2_ref[0], preferred_element_type=jnp.float32) + b2_ref[0]
    gpe = gpe_ref[...]
    ii = jax.lax.broadcasted_iota(jnp.int32, gpe.shape, 1)
    g = jnp.sum(jnp.where(ii == e, gpe, 0.0), axis=1, keepdims=True)
    acc_ref[...] += g * y

    @pl.when(e == E - 1)
    def _():
        o_ref[...] = _ln(x + acc_ref[...], g2_ref[...], bb2_ref[...])


def kernel(src, in_proj_w, in_proj_b, out_proj_w, out_proj_b, ln1_g, ln1_b,
           ln2_g, ln2_b, router_w, router_b, w1, b1, w2, b2):
    Bq, T, D = src.shape
    dh = D // H
    F = w1.shape[2]
    x2 = src.reshape(T, D)

    bt = min(512, T)
    qkv = pl.pallas_call(
        _qkv_kernel,
        grid=(T // bt,),
        in_specs=[
            pl.BlockSpec((bt, D), lambda i: (i, 0)),
            pl.BlockSpec((D, 3 * D), lambda i: (0, 0)),
            pl.BlockSpec((1, 3 * D), lambda i: (0, 0)),
        ],
        out_specs=pl.BlockSpec((bt, 3 * D), lambda i: (i, 0)),
        out_shape=jax.ShapeDtypeStruct((T, 3 * D), jnp.float32),
    )(x2, in_proj_w.T, in_proj_b.reshape(1, 3 * D))

    bq = min(512, T)
    ctx = pl.pallas_call(
        functools.partial(_attn_kernel, scale=1.0 / (dh ** 0.5), dh=dh),
        grid=(T // bq,),
        in_specs=[
            pl.BlockSpec((bq, 3 * D), lambda i: (i, 0)),
            pl.BlockSpec((T, 3 * D), lambda i: (0, 0)),
        ],
        out_specs=pl.BlockSpec((bq, D), lambda i: (i, 0)),
        out_shape=jax.ShapeDtypeStruct((T, D), jnp.float32),
    )(qkv, qkv)

    bp = min(512, T)
    x, gpe = pl.pallas_call(
        _post_attn_kernel,
        grid=(T // bp,),
        in_specs=[
            pl.BlockSpec((bp, D), lambda i: (i, 0)),
            pl.BlockSpec((bp, D), lambda i: (i, 0)),
            pl.BlockSpec((D, D), lambda i: (0, 0)),
            pl.BlockSpec((1, D), lambda i: (0, 0)),
            pl.BlockSpec((1, D), lambda i: (0, 0)),
            pl.BlockSpec((1, D), lambda i: (0, 0)),
            pl.BlockSpec((D, E), lambda i: (0, 0)),
            pl.BlockSpec((1, E), lambda i: (0, 0)),
        ],
        out_specs=[
            pl.BlockSpec((bp, D), lambda i: (i, 0)),
            pl.BlockSpec((bp, E), lambda i: (i, 0)),
        ],
        out_shape=[
            jax.ShapeDtypeStruct((T, D), jnp.float32),
            jax.ShapeDtypeStruct((T, E), jnp.float32),
        ],
    )(ctx, x2, out_proj_w.T, out_proj_b.reshape(1, D),
      ln1_g.reshape(1, D), ln1_b.reshape(1, D),
      router_w, router_b.reshape(1, E))

    bm = min(1024, T)
    out = pl.pallas_call(
        _moe_kernel,
        grid=(T // bm, E),
        in_specs=[
            pl.BlockSpec((bm, D), lambda i, e: (i, 0)),
            pl.BlockSpec((bm, E), lambda i, e: (i, 0)),
            pl.BlockSpec((1, D, F), lambda i, e: (e, 0, 0)),
            pl.BlockSpec((1, 1, F), lambda i, e: (e, 0, 0)),
            pl.BlockSpec((1, F, D), lambda i, e: (e, 0, 0)),
            pl.BlockSpec((1, 1, D), lambda i, e: (e, 0, 0)),
            pl.BlockSpec((1, D), lambda i, e: (0, 0)),
            pl.BlockSpec((1, D), lambda i, e: (0, 0)),
        ],
        out_specs=pl.BlockSpec((bm, D), lambda i, e: (i, 0)),
        out_shape=jax.ShapeDtypeStruct((T, D), jnp.float32),
        scratch_shapes=[pltpu.VMEM((bm, D), jnp.float32)],
    )(x, gpe, w1, b1.reshape(E, 1, F), w2, b2.reshape(E, 1, D),
      ln2_g.reshape(1, D), ln2_b.reshape(1, D))

    return out.reshape(Bq, T, D)
